# Initial kernel scaffold; baseline (speedup 1.0000x reference)
import functools

import jax
import jax.numpy as jnp
from jax import lax
from jax.experimental import pallas as pl
from jax.experimental.pallas import tpu as pltpu
from jax.experimental.pallas import tpu_sc as plsc

_SCALE = 8.0

_CHUNK = 128
_RING = 4


def _scale_body(w_ref, o_ref):
    o_ref[...] = w_ref[...] * _SCALE


def _scale_table(w):
    v, d = w.shape
    blk = 10000
    assert v % blk == 0 and blk % 8 == 0
    return pl.pallas_call(
        _scale_body,
        out_shape=jax.ShapeDtypeStruct((v, d), w.dtype),
        grid=(v // blk,),
        in_specs=[pl.BlockSpec((blk, d), lambda i: (i, 0))],
        out_specs=pl.BlockSpec((blk, d), lambda i: (i, 0)),
    )(w)


@functools.partial(jax.jit, static_argnums=(2, 3))
def _sc_gather(idx, table, n, d):
    info = plsc.get_sparse_core_info()
    nw = info.num_cores * info.num_subcores
    pw = n // nw
    nch = pw // _CHUNK
    ng = nch // _RING
    assert pw * nw == n and nch * _CHUNK == pw and ng * _RING == nch

    mesh = plsc.VectorSubcoreMesh(core_axis_name="c", subcore_axis_name="s")

    @functools.partial(
        pl.kernel,
        mesh=mesh,
        out_type=jax.ShapeDtypeStruct((n, d), jnp.float32),
        scratch_types=(
            [pltpu.VMEM((pw,), jnp.int32)]
            + [pltpu.VMEM((_CHUNK, d), jnp.float32) for _ in range(_RING)]
            + [pltpu.SemaphoreType.DMA for _ in range(_RING)]
        ),
    )
    def k(idx_hbm, table_hbm, out_hbm, idx_v, *rest):
        rows = rest[:_RING]
        sems = rest[_RING:]
        wid = lax.axis_index("s") * info.num_cores + lax.axis_index("c")
        base = wid * pw
        pltpu.sync_copy(idx_hbm.at[pl.ds(base, pw)], idx_v)
        for b in range(_RING):
            pltpu.async_copy(
                table_hbm.at[idx_v.at[pl.ds(b * _CHUNK, _CHUNK)]],
                rows[b], sems[b])

        def body(g, carry):
            for b in range(_RING):
                j = g * _RING + b
                pltpu.make_async_copy(
                    table_hbm.at[idx_v.at[pl.ds(j * _CHUNK, _CHUNK)]],
                    rows[b], sems[b]).wait()
                pltpu.sync_copy(
                    rows[b], out_hbm.at[pl.ds(base + j * _CHUNK, _CHUNK)])
                nj = j + _RING

                @pl.when(nj < nch)
                def _():
                    pltpu.async_copy(
                        table_hbm.at[idx_v.at[pl.ds(nj * _CHUNK, _CHUNK)]],
                        rows[b], sems[b])
            return carry

        lax.fori_loop(0, ng, body, None)

    return k(idx, table)


def kernel(tokens, word_vectors):
    b, l = tokens.shape
    v, d = word_vectors.shape
    scaled = _scale_table(word_vectors)
    out = _sc_gather(tokens.reshape(-1), scaled, b * l, d)
    return out.reshape(b, l, d)


# speedup vs baseline: 3.9632x; 3.9632x over previous
import functools

import jax
import jax.numpy as jnp
from jax import lax
from jax.experimental import pallas as pl
from jax.experimental.pallas import tpu as pltpu
from jax.experimental.pallas import tpu_sc as plsc

_SCALE = 8.0

_CHUNK = 128
_RING = 4


def _scale_body(w_ref, o_ref):
    o_ref[...] = w_ref[...] * _SCALE


def _scale_table(w):
    v, d = w.shape
    blk = 10000
    assert v % blk == 0 and blk % 8 == 0
    return pl.pallas_call(
        _scale_body,
        out_shape=jax.ShapeDtypeStruct((v, d), w.dtype),
        grid=(v // blk,),
        in_specs=[pl.BlockSpec((blk, d), lambda i: (i, 0))],
        out_specs=pl.BlockSpec((blk, d), lambda i: (i, 0)),
    )(w)


@functools.partial(jax.jit, static_argnums=(2, 3))
def _sc_gather(idx, table, n, d):
    info = plsc.get_sparse_core_info()
    nw = info.num_cores * info.num_subcores
    pw = n // nw
    nch = pw // _CHUNK
    ng = nch // _RING
    assert pw * nw == n and nch * _CHUNK == pw and ng * _RING == nch

    mesh = plsc.VectorSubcoreMesh(core_axis_name="c", subcore_axis_name="s")

    @functools.partial(
        pl.kernel,
        mesh=mesh,
        compiler_params=pltpu.CompilerParams(use_tc_tiling_on_sc=False),
        out_type=jax.ShapeDtypeStruct((n, d), jnp.float32),
        scratch_types=(
            [pltpu.VMEM((pw,), jnp.int32)]
            + [pltpu.VMEM((_CHUNK, d), jnp.float32) for _ in range(_RING)]
            + [pltpu.SemaphoreType.DMA for _ in range(_RING)]
        ),
    )
    def k(idx_hbm, table_hbm, out_hbm, idx_v, *rest):
        rows = rest[:_RING]
        sems = rest[_RING:]
        wid = lax.axis_index("s") * info.num_cores + lax.axis_index("c")
        base = wid * pw
        pltpu.sync_copy(idx_hbm.at[pl.ds(base, pw)], idx_v)
        for b in range(_RING):
            pltpu.async_copy(
                table_hbm.at[idx_v.at[pl.ds(b * _CHUNK, _CHUNK)]],
                rows[b], sems[b])

        def body(g, carry):
            for b in range(_RING):
                j = g * _RING + b
                pltpu.make_async_copy(
                    table_hbm.at[idx_v.at[pl.ds(j * _CHUNK, _CHUNK)]],
                    rows[b], sems[b]).wait()
                pltpu.sync_copy(
                    rows[b], out_hbm.at[pl.ds(base + j * _CHUNK, _CHUNK)])
                nj = j + _RING

                @pl.when(nj < nch)
                def _():
                    pltpu.async_copy(
                        table_hbm.at[idx_v.at[pl.ds(nj * _CHUNK, _CHUNK)]],
                        rows[b], sems[b])
            return carry

        lax.fori_loop(0, ng, body, None)

    return k(idx, table)


def kernel(tokens, word_vectors):
    b, l = tokens.shape
    v, d = word_vectors.shape
    scaled = _scale_table(word_vectors)
    out = _sc_gather(tokens.reshape(-1), scaled, b * l, d)
    return out.reshape(b, l, d)


# speedup vs baseline: 3.9655x; 1.0006x over previous
import functools

import jax
import jax.numpy as jnp
from jax import lax
from jax.experimental import pallas as pl
from jax.experimental.pallas import tpu as pltpu
from jax.experimental.pallas import tpu_sc as plsc

_SCALE = 8.0

_RING = 4
_SPLIT = 120


def _scale_body(w_ref, o_ref):
    o_ref[...] = w_ref[...] * _SCALE


def _scale_table(w):
    v, d = w.shape
    blk = 10000
    assert v % blk == 0 and blk % 8 == 0
    return pl.pallas_call(
        _scale_body,
        out_shape=jax.ShapeDtypeStruct((v, d), w.dtype),
        grid=(v // blk,),
        in_specs=[pl.BlockSpec((blk, d), lambda i: (i, 0))],
        out_specs=pl.BlockSpec((blk, d), lambda i: (i, 0)),
    )(w)


@functools.partial(jax.jit, static_argnums=(2, 3, 4))
def _sc_gather(idx, table, b, l, d):
    info = plsc.get_sparse_core_info()
    nw = info.num_cores * info.num_subcores
    pb = b // nw
    ng = pb // _RING
    assert pb * nw == b and ng * _RING == pb

    mesh = plsc.VectorSubcoreMesh(core_axis_name="c", subcore_axis_name="s")

    @functools.partial(
        pl.kernel,
        mesh=mesh,
        compiler_params=pltpu.CompilerParams(use_tc_tiling_on_sc=False),
        out_type=jax.ShapeDtypeStruct((b, l, d), jnp.float32),
        scratch_types=(
            [pltpu.VMEM((pb * l,), jnp.int32)]
            + [pltpu.VMEM((l, d), jnp.float32) for _ in range(_RING)]
            + [pltpu.SemaphoreType.DMA for _ in range(2 * _RING)]
        ),
    )
    def k(idx_hbm, table_hbm, out_hbm, idx_v, *rest):
        rows = rest[:_RING]
        sems = rest[_RING:]
        wid = lax.axis_index("s") * info.num_cores + lax.axis_index("c")
        base = wid * pb
        pltpu.sync_copy(idx_hbm.at[pl.ds(base * l, pb * l)], idx_v)

        def fire(i, r):
            pltpu.async_copy(
                table_hbm.at[idx_v.at[pl.ds(i * l, _SPLIT)]],
                rows[r].at[pl.ds(0, _SPLIT)], sems[2 * r])
            pltpu.async_copy(
                table_hbm.at[idx_v.at[pl.ds(i * l + _SPLIT, l - _SPLIT)]],
                rows[r].at[pl.ds(_SPLIT, l - _SPLIT)], sems[2 * r + 1])

        def drain(i, r):
            pltpu.make_async_copy(
                table_hbm.at[idx_v.at[pl.ds(i * l, _SPLIT)]],
                rows[r].at[pl.ds(0, _SPLIT)], sems[2 * r]).wait()
            pltpu.make_async_copy(
                table_hbm.at[idx_v.at[pl.ds(i * l + _SPLIT, l - _SPLIT)]],
                rows[r].at[pl.ds(_SPLIT, l - _SPLIT)], sems[2 * r + 1]).wait()

        for r in range(_RING):
            fire(r, r)

        def body(g, carry):
            for r in range(_RING):
                i = g * _RING + r
                drain(i, r)
                pltpu.sync_copy(rows[r], out_hbm.at[base + i])
                ni = i + _RING

                @pl.when(ni < pb)
                def _():
                    fire(ni, r)
            return carry

        lax.fori_loop(0, ng, body, None)

    return k(idx, table)


def kernel(tokens, word_vectors):
    b, l = tokens.shape
    v, d = word_vectors.shape
    scaled = _scale_table(word_vectors)
    return _sc_gather(tokens.reshape(-1), scaled, b, l, d)


# speedup vs baseline: 4.5108x; 1.1375x over previous
import functools

import jax
import jax.numpy as jnp
from jax import lax
from jax.experimental import pallas as pl
from jax.experimental.pallas import tpu as pltpu
from jax.experimental.pallas import tpu_sc as plsc

_SCALE = 8.0

_RING = 2
_SPLIT = 120


def _scale_pad_body(w_ref, o_ref):
    w = w_ref[...]
    o_ref[...] = jnp.concatenate([w * _SCALE, jnp.zeros_like(w)], axis=1)


def _scale_pad_table(w):
    v, d = w.shape
    blk = 10000
    assert v % blk == 0 and blk % 8 == 0
    return pl.pallas_call(
        _scale_pad_body,
        out_shape=jax.ShapeDtypeStruct((v, 2 * d), w.dtype),
        grid=(v // blk,),
        in_specs=[pl.BlockSpec((blk, d), lambda i: (i, 0))],
        out_specs=pl.BlockSpec((blk, 2 * d), lambda i: (i, 0)),
    )(w)


@functools.partial(jax.jit, static_argnums=(2, 3, 4))
def _sc_gather(idx, table, b, l, d):
    info = plsc.get_sparse_core_info()
    nl = info.num_lanes
    nw = info.num_cores * info.num_subcores
    pb = b // nw
    ng = pb // _RING
    assert pb * nw == b and ng * _RING == pb

    mesh = plsc.VectorSubcoreMesh(core_axis_name="c", subcore_axis_name="s")

    @functools.partial(
        pl.kernel,
        mesh=mesh,
        out_type=jax.ShapeDtypeStruct((b, l, d), jnp.float32),
        scratch_types=(
            [pltpu.VMEM((pb * l,), jnp.int32)]
            + [pltpu.VMEM((l, 2 * d), jnp.float32) for _ in range(_RING)]
            + [pltpu.VMEM((l, d), jnp.float32)]
            + [pltpu.SemaphoreType.DMA for _ in range(2 * _RING)]
        ),
    )
    def k(idx_hbm, table_hbm, out_hbm, idx_v, *rest):
        rows = rest[:_RING]
        wbuf = rest[_RING]
        sems = rest[_RING + 1:]
        wid = lax.axis_index("s") * info.num_cores + lax.axis_index("c")
        base = wid * pb
        pltpu.sync_copy(idx_hbm.at[pl.ds(base * l, pb * l)], idx_v)

        def fire(i, r):
            pltpu.async_copy(
                table_hbm.at[idx_v.at[pl.ds(i * l, _SPLIT)]],
                rows[r].at[pl.ds(0, _SPLIT)], sems[2 * r])
            pltpu.async_copy(
                table_hbm.at[idx_v.at[pl.ds(i * l + _SPLIT, l - _SPLIT)]],
                rows[r].at[pl.ds(_SPLIT, l - _SPLIT)], sems[2 * r + 1])

        def drain(i, r):
            pltpu.make_async_copy(
                table_hbm.at[idx_v.at[pl.ds(i * l, _SPLIT)]],
                rows[r].at[pl.ds(0, _SPLIT)], sems[2 * r]).wait()
            pltpu.make_async_copy(
                table_hbm.at[idx_v.at[pl.ds(i * l + _SPLIT, l - _SPLIT)]],
                rows[r].at[pl.ds(_SPLIT, l - _SPLIT)], sems[2 * r + 1]).wait()

        def repack(r):
            def rows8(t, carry):
                for rr in range(8):
                    for j in range(d // nl):
                        wbuf[t * 8 + rr, pl.ds(j * nl, nl)] = (
                            rows[r][t * 8 + rr, pl.ds(j * nl, nl)])
                return carry
            lax.fori_loop(0, l // 8, rows8, None)

        for r in range(_RING):
            fire(r, r)

        def body(g, carry):
            for r in range(_RING):
                i = g * _RING + r
                drain(i, r)
                repack(r)
                pltpu.sync_copy(wbuf, out_hbm.at[base + i])
                ni = i + _RING

                @pl.when(ni < pb)
                def _():
                    fire(ni, r)
            return carry

        lax.fori_loop(0, ng, body, None)

    return k(idx, table)


def kernel(tokens, word_vectors):
    b, l = tokens.shape
    v, d = word_vectors.shape
    scaled = _scale_pad_table(word_vectors)
    return _sc_gather(tokens.reshape(-1), scaled, b, l, d)
